# baseline (speedup 1.0000x reference)
import functools

import jax
import jax.numpy as jnp
from jax import lax
from jax.experimental import pallas as pl
from jax.experimental.pallas import tpu as pltpu
from jax.experimental.pallas import tpu_sc as plsc

B, S, H, V = 4, 2048, 1024, 50257
NC, NS = 2, 16
NW = NC * NS
SEQ_PER_W = S // NW
P = 8
NCH = SEQ_PER_W // P
NBUF = 3
LANES = 16
UNROLL = 8


def _emb_body(x_hbm, tab_hbm, pos_hbm, out_hbm, idx_v, sb_v,
              isem0, isem1, isem2, osem0, osem1, osem2):
    wid = lax.axis_index("s") * NC + lax.axis_index("c")
    s0 = wid * SEQ_PER_W
    isems = (isem0, isem1, isem2)
    osems = (osem0, osem1, osem2)

    for b in range(B):
        pltpu.sync_copy(x_hbm.at[b, pl.ds(s0, SEQ_PER_W)],
                        idx_v.at[pl.ds(b * SEQ_PER_W, SEQ_PER_W)])

    def in_descs(c, buf):
        d = [pltpu.make_async_copy(pos_hbm.at[pl.ds(s0 + c * P, P)],
                                   sb_v.at[buf, B], isems[buf])]
        for b in range(B):
            d.append(pltpu.make_async_copy(
                tab_hbm.at[idx_v.at[pl.ds(b * SEQ_PER_W + c * P, P)]],
                sb_v.at[buf, b], isems[buf]))
        return d

    def out_descs(c, buf):
        return [pltpu.make_async_copy(
                    sb_v.at[buf, b],
                    out_hbm.at[b, pl.ds(s0 + c * P, P)], osems[buf])
                for b in range(B)]

    def start(c, buf):
        for d in in_descs(c, buf):
            d.start()

    start(0, 0)
    start(1, 1)

    for c in range(NCH):
        p = c % NBUF
        for d in in_descs(c, p):
            d.wait()

        @plsc.parallel_loop(0, P * (H // LANES), unroll=UNROLL)
        def _(k):
            r = lax.shift_right_logical(k, 6)
            off = pl.multiple_of(
                lax.shift_left(lax.bitwise_and(k, 63), 4), LANES)
            sl = pl.ds(off, LANES)
            pe = sb_v[p, B, r, sl]
            for b in range(B):
                sb_v[p, b, r, sl] = sb_v[p, b, r, sl] + pe

        for d in out_descs(c, p):
            d.start()

        if c + 2 < NCH:
            q = (c + 2) % NBUF
            if c >= 1:
                for d in out_descs(c - 1, q):
                    d.wait()
            start(c + 2, q)

    for c in (NCH - 3, NCH - 2, NCH - 1):
        for d in out_descs(c, c % NBUF):
            d.wait()


@jax.jit
def _emb(x2d, table, pos):
    mesh = plsc.VectorSubcoreMesh(core_axis_name="c", subcore_axis_name="s")
    f = functools.partial(
        pl.kernel,
        mesh=mesh,
        out_type=jax.ShapeDtypeStruct((B, S, H), jnp.float32),
        scratch_types=[
            pltpu.VMEM((B * SEQ_PER_W,), jnp.int32),
            pltpu.VMEM((NBUF, B + 1, P, H), jnp.float32),
            pltpu.SemaphoreType.DMA,
            pltpu.SemaphoreType.DMA,
            pltpu.SemaphoreType.DMA,
            pltpu.SemaphoreType.DMA,
            pltpu.SemaphoreType.DMA,
            pltpu.SemaphoreType.DMA,
        ],
    )(_emb_body)
    return f(x2d, table, pos)


def kernel(x, token_table, pos_emb):
    pos = pos_emb.reshape(S, H)
    return _emb(x.astype(jnp.int32), token_table, pos)


# speedup vs baseline: 1.0196x; 1.0196x over previous
import functools

import jax
import jax.numpy as jnp
from jax import lax
from jax.experimental import pallas as pl
from jax.experimental.pallas import tpu as pltpu
from jax.experimental.pallas import tpu_sc as plsc

B, S, H, V = 4, 2048, 1024, 50257
NC, NS = 2, 16
NW = NC * NS
SEQ_PER_W = S // NW
P = 8
NCH = SEQ_PER_W // P
LANES = 16
UNROLL = 8


def _emb_body(x_hbm, tab_hbm, pos_hbm, out_hbm, idx_v, sb_v, pos_v, sem0, sem1):
    wid = lax.axis_index("s") * NC + lax.axis_index("c")
    s0 = wid * SEQ_PER_W
    sems = (sem0, sem1)

    for b in range(B):
        pltpu.sync_copy(x_hbm.at[b, pl.ds(s0, SEQ_PER_W)],
                        idx_v.at[pl.ds(b * SEQ_PER_W, SEQ_PER_W)])

    def descs(c, buf):
        d = [pltpu.make_async_copy(pos_hbm.at[pl.ds(s0 + c * P, P)],
                                   pos_v.at[buf], sems[buf])]
        for b in range(B):
            d.append(pltpu.make_async_copy(
                tab_hbm.at[idx_v.at[pl.ds(b * SEQ_PER_W + c * P, P)]],
                sb_v.at[buf, b], sems[buf]))
        return d

    def start(c, buf):
        for d in descs(c, buf):
            d.start()

    start(0, 0)
    start(1, 1)

    def pair_body(i, _):
        for sub in range(2):
            c = 2 * i + sub
            buf = sub
            for d in descs(c, buf):
                d.wait()

            @plsc.parallel_loop(0, P * (H // LANES), unroll=UNROLL)
            def _(k):
                r = lax.shift_right_logical(k, 6)
                off = pl.multiple_of(
                    lax.shift_left(lax.bitwise_and(k, 63), 4), LANES)
                sl = pl.ds(off, LANES)
                p = pos_v[buf, r, sl]
                for b in range(B):
                    sb_v[buf, b, r, sl] = sb_v[buf, b, r, sl] + p

            for b in range(B):
                pltpu.sync_copy(sb_v.at[buf, b],
                                out_hbm.at[b, pl.ds(s0 + c * P, P)])

            @pl.when(c + 2 < NCH)
            def _():
                start(c + 2, buf)
        return 0

    lax.fori_loop(0, NCH // 2, pair_body, 0)


@jax.jit
def _emb(x2d, table, pos):
    mesh = plsc.VectorSubcoreMesh(core_axis_name="c", subcore_axis_name="s")
    f = functools.partial(
        pl.kernel,
        mesh=mesh,
        out_type=jax.ShapeDtypeStruct((B, S, H), jnp.float32),
        scratch_types=[
            pltpu.VMEM((B * SEQ_PER_W,), jnp.int32),
            pltpu.VMEM((2, B, P, H), jnp.float32),
            pltpu.VMEM((2, P, H), jnp.float32),
            pltpu.SemaphoreType.DMA,
            pltpu.SemaphoreType.DMA,
        ],
    )(_emb_body)
    return f(x2d, table, pos)


def kernel(x, token_table, pos_emb):
    pos = pos_emb.reshape(S, H)
    return _emb(x.astype(jnp.int32), token_table, pos)


# speedup vs baseline: 1.0262x; 1.0065x over previous
import functools

import jax
import jax.numpy as jnp
from jax import lax
from jax.experimental import pallas as pl
from jax.experimental.pallas import tpu as pltpu
from jax.experimental.pallas import tpu_sc as plsc

B, S, H, V = 4, 2048, 1024, 50257
NC, NS = 2, 16
NW = NC * NS
SEQ_PER_W = S // NW
P = 8
NCH = SEQ_PER_W // P
LANES = 16
UNROLL = 8


def _emb_body(x_hbm, tab_hbm, pos_hbm, out_hbm, idx_v, sb_v, pos_v, sem0, sem1):
    wid = lax.axis_index("s") * NC + lax.axis_index("c")
    s0 = wid * SEQ_PER_W
    sems = (sem0, sem1)

    for b in range(B):
        pltpu.sync_copy(x_hbm.at[b, pl.ds(s0, SEQ_PER_W)],
                        idx_v.at[pl.ds(b * SEQ_PER_W, SEQ_PER_W)])

    def descs(c, buf):
        d = [pltpu.make_async_copy(pos_hbm.at[pl.ds(s0 + c * P, P)],
                                   pos_v.at[buf], sems[buf])]
        for b in range(B):
            d.append(pltpu.make_async_copy(
                tab_hbm.at[idx_v.at[pl.ds(b * SEQ_PER_W + c * P, P)]],
                sb_v.at[buf, b], sems[buf]))
        return d

    def start(c, buf):
        for d in descs(c, buf):
            d.start()

    start(0, 0)
    start(1, 1)

    def pair_body(i, _):
        for sub in range(2):
            c = 2 * i + sub
            buf = sub
            for d in descs(c, buf):
                d.wait()

            @plsc.parallel_loop(0, P * (H // LANES), unroll=UNROLL)
            def _(k):
                r = lax.shift_right_logical(k, 6)
                off = pl.multiple_of(
                    lax.shift_left(lax.bitwise_and(k, 63), 4), LANES)
                sl = pl.ds(off, LANES)
                p = pos_v[buf, r, sl]
                for b in range(B):
                    plsc.addupdate(sb_v.at[buf, b, r, sl], p)

            for b in range(B):
                pltpu.sync_copy(sb_v.at[buf, b],
                                out_hbm.at[b, pl.ds(s0 + c * P, P)])

            @pl.when(c + 2 < NCH)
            def _():
                start(c + 2, buf)
        return 0

    lax.fori_loop(0, NCH // 2, pair_body, 0)


@jax.jit
def _emb(x2d, table, pos):
    mesh = plsc.VectorSubcoreMesh(core_axis_name="c", subcore_axis_name="s")
    f = functools.partial(
        pl.kernel,
        mesh=mesh,
        out_type=jax.ShapeDtypeStruct((B, S, H), jnp.float32),
        scratch_types=[
            pltpu.VMEM((B * SEQ_PER_W,), jnp.int32),
            pltpu.VMEM((2, B, P, H), jnp.float32),
            pltpu.VMEM((2, P, H), jnp.float32),
            pltpu.SemaphoreType.DMA,
            pltpu.SemaphoreType.DMA,
        ],
    )(_emb_body)
    return f(x2d, table, pos)


def kernel(x, token_table, pos_emb):
    pos = pos_emb.reshape(S, H)
    return _emb(x.astype(jnp.int32), token_table, pos)


# speedup vs baseline: 1.0329x; 1.0065x over previous
import functools

import jax
import jax.numpy as jnp
from jax import lax
from jax.experimental import pallas as pl
from jax.experimental.pallas import tpu as pltpu
from jax.experimental.pallas import tpu_sc as plsc

B, S, H, V = 4, 2048, 1024, 50257
NC, NS = 2, 16
NW = NC * NS
SEQ_PER_W = S // NW
P = 8
HP = P // 2
NCH = SEQ_PER_W // P
LANES = 16
UNROLL = 8


def _emb_body(x_hbm, tab_hbm, pos_hbm, out_hbm, idx_v, sb_v, pos_v,
              isem0, isem1, osem0, osem1):
    wid = lax.axis_index("s") * NC + lax.axis_index("c")
    s0 = wid * SEQ_PER_W
    isems = (isem0, isem1)
    osems = (osem0, osem1)

    for b in range(B):
        pltpu.sync_copy(x_hbm.at[b, pl.ds(s0, SEQ_PER_W)],
                        idx_v.at[pl.ds(b * SEQ_PER_W, SEQ_PER_W)])

    def in_descs(c, buf):
        d = [pltpu.make_async_copy(pos_hbm.at[pl.ds(s0 + c * P, P)],
                                   pos_v.at[buf], isems[buf])]
        for b in range(B):
            d.append(pltpu.make_async_copy(
                tab_hbm.at[idx_v.at[pl.ds(b * SEQ_PER_W + c * P, P)]],
                sb_v.at[buf, b], isems[buf]))
        return d

    def half_out_descs(c, buf, half):
        return [pltpu.make_async_copy(
                    sb_v.at[buf, b, pl.ds(half * HP, HP)],
                    out_hbm.at[b, pl.ds(s0 + c * P + half * HP, HP)],
                    osems[buf])
                for b in range(B)]

    def start(descs):
        for d in descs:
            d.start()

    def add_half(buf, half):
        @plsc.parallel_loop(0, HP * (H // LANES), unroll=UNROLL)
        def _(k):
            r = half * HP + lax.shift_right_logical(k, 6)
            off = pl.multiple_of(
                lax.shift_left(lax.bitwise_and(k, 63), 4), LANES)
            sl = pl.ds(off, LANES)
            p = pos_v[buf, r, sl]
            for b in range(B):
                plsc.addupdate(sb_v.at[buf, b, r, sl], p)

    start(in_descs(0, 0))

    def pair_body(i, _):
        for sub in range(2):
            c = 2 * i + sub
            buf = sub
            obuf = 1 - sub

            @pl.when(c + 1 < NCH)
            def _():
                @pl.when(c >= 1)
                def _():
                    for half in range(2):
                        for d in half_out_descs(c - 1, obuf, half):
                            d.wait()

                start(in_descs(c + 1, obuf))

            for d in in_descs(c, buf):
                d.wait()

            add_half(buf, 0)
            start(half_out_descs(c, buf, 0))
            add_half(buf, 1)
            start(half_out_descs(c, buf, 1))
        return 0

    lax.fori_loop(0, NCH // 2, pair_body, 0)

    for c in (NCH - 2, NCH - 1):
        for half in range(2):
            for d in half_out_descs(c, c % 2, half):
                d.wait()


@jax.jit
def _emb(x2d, table, pos):
    mesh = plsc.VectorSubcoreMesh(core_axis_name="c", subcore_axis_name="s")
    f = functools.partial(
        pl.kernel,
        mesh=mesh,
        out_type=jax.ShapeDtypeStruct((B, S, H), jnp.float32),
        scratch_types=[
            pltpu.VMEM((B * SEQ_PER_W,), jnp.int32),
            pltpu.VMEM((2, B, P, H), jnp.float32),
            pltpu.VMEM((2, P, H), jnp.float32),
            pltpu.SemaphoreType.DMA,
            pltpu.SemaphoreType.DMA,
            pltpu.SemaphoreType.DMA,
            pltpu.SemaphoreType.DMA,
        ],
    )(_emb_body)
    return f(x2d, table, pos)


def kernel(x, token_table, pos_emb):
    pos = pos_emb.reshape(S, H)
    return _emb(x.astype(jnp.int32), token_table, pos)


# speedup vs baseline: 1.0534x; 1.0199x over previous
import functools

import jax
import jax.numpy as jnp
from jax import lax
from jax.experimental import pallas as pl
from jax.experimental.pallas import tpu as pltpu
from jax.experimental.pallas import tpu_sc as plsc

B, S, H, V = 4, 2048, 1024, 50257
NC, NS = 2, 16
NW = NC * NS
SEQ_PER_W = S // NW
P = 8
NCH = SEQ_PER_W // P
ROWS = B * P
LANES = 16
UNROLL = 8


def _emb_body(x_hbm, tab_hbm, pos_hbm, out_hbm, idx_v, sb_v, pos_v,
              isem0, isem1, osem0, osem1):
    wid = lax.axis_index("s") * NC + lax.axis_index("c")
    s0 = wid * SEQ_PER_W
    isems = (isem0, isem1)
    osems = (osem0, osem1)

    pltpu.sync_copy(x_hbm.at[wid], idx_v)

    def in_descs(c, buf):
        return [
            pltpu.make_async_copy(pos_hbm.at[pl.ds(s0 + c * P, P)],
                                  pos_v.at[buf], isems[buf]),
            pltpu.make_async_copy(tab_hbm.at[idx_v.at[c]],
                                  sb_v.at[buf], isems[buf]),
        ]

    def out_descs(c, buf):
        return [pltpu.make_async_copy(
                    sb_v.at[buf, pl.ds(b * P, P)],
                    out_hbm.at[b, pl.ds(s0 + c * P, P)], osems[buf])
                for b in range(B)]

    def start(descs):
        for d in descs:
            d.start()

    start(in_descs(0, 0))

    def pair_body(i, _):
        for sub in range(2):
            c = 2 * i + sub
            buf = sub
            obuf = 1 - sub

            @pl.when(c + 1 < NCH)
            def _():
                @pl.when(c >= 1)
                def _():
                    for d in out_descs(c - 1, obuf):
                        d.wait()

                start(in_descs(c + 1, obuf))

            for d in in_descs(c, buf):
                d.wait()

            @plsc.parallel_loop(0, P * (H // LANES), unroll=UNROLL)
            def _(k):
                r = lax.shift_right_logical(k, 6)
                off = pl.multiple_of(
                    lax.shift_left(lax.bitwise_and(k, 63), 4), LANES)
                sl = pl.ds(off, LANES)
                p = pos_v[buf, r, sl]
                for b in range(B):
                    plsc.addupdate(sb_v.at[buf, b * P + r, sl], p)

            start(out_descs(c, buf))
        return 0

    lax.fori_loop(0, NCH // 2, pair_body, 0)

    for c in (NCH - 2, NCH - 1):
        for d in out_descs(c, c % 2):
            d.wait()


@jax.jit
def _emb(x2d, table, pos):
    mesh = plsc.VectorSubcoreMesh(core_axis_name="c", subcore_axis_name="s")
    f = functools.partial(
        pl.kernel,
        mesh=mesh,
        out_type=jax.ShapeDtypeStruct((B, S, H), jnp.float32),
        scratch_types=[
            pltpu.VMEM((NCH, ROWS), jnp.int32),
            pltpu.VMEM((2, ROWS, H), jnp.float32),
            pltpu.VMEM((2, P, H), jnp.float32),
            pltpu.SemaphoreType.DMA,
            pltpu.SemaphoreType.DMA,
            pltpu.SemaphoreType.DMA,
            pltpu.SemaphoreType.DMA,
        ],
    )(_emb_body)
    return f(x2d, table, pos)


def kernel(x, token_table, pos_emb):
    pos = pos_emb.reshape(S, H)
    x2 = (x.astype(jnp.int32)
          .reshape(B, NW, NCH, P)
          .transpose(1, 2, 0, 3)
          .reshape(NW, NCH, ROWS))
    return _emb(x2, token_table, pos)
